# baseline (speedup 1.0000x reference)
import jax
import jax.numpy as jnp
from jax import lax
from jax.experimental import pallas as pl
from jax.experimental.pallas import tpu as pltpu
from jax.experimental.pallas import tpu_sc as plsc

N_ROWS = 1_000_000
DIM = 16
BATCH = 16384
NC = 2
NS = 16
NW = NC * NS
BPW = BATCH // NW
MW = 128
SW = 512
NG = (N_ROWS + SW - 1) // SW
NSLAB = 60
CAP = 576


def _rsqrt(s):
    i = plsc.bitcast(s, jnp.int32)
    y = plsc.bitcast(jnp.int32(0x5F3759DF) - (i >> 1), jnp.float32)
    for _ in range(3):
        y = y * (1.5 - 0.5 * s * y * y)
    return y


def _prep_body(zLt_hbm, zTt_hbm, a_hbm, msg_hbm, z_v, m_v, a_v):
    wid = lax.axis_index("s") * NC + lax.axis_index("c")
    base = wid * BPW

    pltpu.sync_copy(a_hbm, a_v)
    one_m_alpha = 1.0 - a_v[...]
    eps = jnp.full((DIM,), 1e-24, jnp.float32)
    iota = lax.iota(jnp.int32, DIM)

    for half, zt_hbm in ((0, zLt_hbm), (1, zTt_hbm)):
        pltpu.sync_copy(zt_hbm.at[:, pl.ds(base, BPW)], z_v)

        def blk(k, _, half=half):
            acc = jnp.zeros((DIM,), jnp.float32)
            rows = []
            for d in range(DIM):
                row = z_v[d, pl.ds(k * DIM, DIM)]
                rows.append(row)
                acc = acc + row * row
            rs = one_m_alpha * _rsqrt(jnp.maximum(acc, eps))
            for d in range(DIM):
                plsc.store_scatter(
                    m_v,
                    [k * DIM + iota, jnp.full((DIM,), half * DIM + d, jnp.int32)],
                    rows[d] * rs)
            return _

        lax.fori_loop(0, BPW // DIM, blk, 0)
    pltpu.sync_copy(m_v, msg_hbm.at[pl.ds(base, BPW)])


_mesh = plsc.VectorSubcoreMesh(core_axis_name="c", subcore_axis_name="s",
                               num_cores=NC, num_subcores=NS)

_prep = pl.kernel(
    _prep_body,
    out_type=jax.ShapeDtypeStruct((BATCH, MW), jnp.float32),
    mesh=_mesh,
    compiler_params=pltpu.CompilerParams(needs_layout_passes=False),
    scratch_types=[
        pltpu.VMEM((DIM, BPW), jnp.float32),
        pltpu.VMEM((BPW, MW), jnp.float32),
        pltpu.VMEM((DIM,), jnp.float32),
    ],
)


def _apply_body(idx_hbm, mLt_hbm, mTt_hbm, msg_hbm, a_hbm, outLt, outTt,
                idx_v, eid_v, val_v, off_v, sord_v,
                msg_v, slab_v, slabe_v, a_v,
                semg0, semg1, semg2, semg3, semp0, semp1, semp2, semp3, semc):
    wid = lax.axis_index("s") * NC + lax.axis_index("c")

    pltpu.sync_copy(a_hbm, a_v)
    alpha = a_v[...]
    eps = jnp.full((DIM,), 1e-24, jnp.float32)
    iota = lax.iota(jnp.int32, DIM)
    zero16 = jnp.zeros((DIM,), jnp.int32)

    ins = (mLt_hbm, mTt_hbm)
    outs = (outLt, outTt)
    semg = (semg0, semg1, semg2, semg3)
    semp = (semp0, semp1, semp2, semp3)

    ICH = 1024

    def clear(b, _):
        eid_v[pl.ds(b * DIM, DIM)] = zero16
        val_v[pl.ds(b * DIM, DIM)] = zero16
        return _

    lax.fori_loop(0, CAP // DIM, clear, 0)

    def chunk(q, cnt):
        pltpu.sync_copy(idx_hbm.at[pl.ds(q * ICH, ICH)], idx_v.at[pl.ds(0, ICH)])

        def scan(b, cnt):
            iv = idx_v[pl.ds(b * DIM, DIM)]
            own = (lax.shift_right_logical(iv, 9) & (NW - 1)) == wid
            e = q * ICH + b * DIM + iota
            c = jnp.minimum(cnt, CAP - DIM)
            plsc.store_compressed(eid_v.at[pl.ds(c, DIM)], e, mask=own)
            plsc.store_compressed(val_v.at[pl.ds(c, DIM)], iv, mask=own)
            npc = plsc.all_reduce_population_count(own)
            return cnt + npc[0]

        return lax.fori_loop(0, ICH // DIM, scan, cnt)

    cnt = lax.fori_loop(0, BATCH // ICH, chunk, jnp.int32(0))
    cnt = jnp.minimum(cnt, CAP)

    NBKT = 63

    def bucket(t, cur):
        plsc.store_scatter(off_v, [jnp.full((DIM,), t, jnp.int32)],
                           jnp.full((DIM,), cur, jnp.int32))

        def sweep(b, cur):
            kv = lax.shift_right_logical(val_v[pl.ds(b * DIM, DIM)], 14)
            m = (kv == t) & ((b * DIM + iota) < cnt)
            c = jnp.minimum(cur, CAP - DIM)
            plsc.store_compressed(sord_v.at[pl.ds(c, DIM)],
                                  b * DIM + iota, mask=m)
            npc = plsc.all_reduce_population_count(m)
            return cur + npc[0]

        return lax.fori_loop(0, CAP // DIM, sweep, cur)

    lax.fori_loop(0, NBKT, bucket, jnp.int32(0))

    gets = []
    for q0 in range(0, CAP, MW):
        qw = min(MW, CAP - q0)
        gets.append(pltpu.async_copy(msg_hbm.at[eid_v.at[pl.ds(q0, qw)]],
                                     msg_v.at[pl.ds(q0, qw)], semc))
    for gd in gets:
        gd.wait()

    def fire_gather(j, slot):
        off = pl.multiple_of((wid + j * NW) * SW, 128)
        pltpu.async_copy(mLt_hbm.at[:, pl.ds(off, SW)],
                         slab_v.at[slot, 0], semg[slot])
        pltpu.async_copy(mTt_hbm.at[:, pl.ds(off, SW)],
                         slab_v.at[slot, 1], semg[slot])

    def drain(sem, slot):
        pltpu.make_async_copy(mLt_hbm.at[:, pl.ds(0, SW)],
                              slab_v.at[slot, 0], sem).wait()
        pltpu.make_async_copy(mLt_hbm.at[:, pl.ds(0, SW)],
                              slab_v.at[slot, 1], sem).wait()

    def apply_updates(slab_ref, lo, hi):
        def upd(i, _):
            ii = jnp.full((DIM,), i, jnp.int32)
            pv = plsc.load_gather(sord_v, [ii])
            col = plsc.load_gather(val_v, [pv]) & (SW - 1)
            for half in (0, 1):
                w = plsc.load_gather(msg_v, [pv, half * DIM + iota])
                g = plsc.load_gather(slab_ref.at[half], [iota, col])
                v = alpha * g + w
                s2 = jnp.maximum(jnp.full((DIM,), jnp.sum(v * v)), eps)
                plsc.store_scatter(slab_ref.at[half], [iota, col],
                                   v * _rsqrt(s2))
            return _

        lax.fori_loop(lo, hi, upd, 0)

    for j0 in (0, 1):
        fire_gather(j0, j0)

    def slab3(m, carry):
        ov = off_v[pl.ds(m * 3, DIM)]
        for s in range(3):
            j = m * 3 + s
            drain(semg[s], s)

            apply_updates(slab_v.at[s], ov[s], ov[s + 1])

            off = pl.multiple_of((wid + j * NW) * SW, 128)
            pltpu.async_copy(slab_v.at[s, 0], outLt.at[:, pl.ds(off, SW)],
                             semp[s])
            pltpu.async_copy(slab_v.at[s, 1], outTt.at[:, pl.ds(off, SW)],
                             semp[s])

            ns = (s + 2) % 3

            @pl.when(j + 2 < NSLAB)
            def _(j=j, ns=ns):
                @pl.when(j >= 1)
                def _():
                    drain(semp[ns], ns)

                fire_gather(j + 2, ns)

        return carry

    lax.fori_loop(0, NSLAB // 3, slab3, 0)
    for s3 in range(3):
        drain(semp[s3], s3)

    ovE = off_v[pl.ds(48, DIM)]

    def sync_slab(goff, lo, hi):
        off = pl.multiple_of(goff * SW, 128)
        for half in (0, 1):
            pltpu.sync_copy(ins[half].at[:, pl.ds(off, SW)],
                            slab_v.at[0, half])
        apply_updates(slab_v.at[0], lo, hi)
        for half in (0, 1):
            pltpu.sync_copy(slab_v.at[0, half], outs[half].at[:, pl.ds(off, SW)])

    sync_slab(wid + 60 * NW, ovE[12], ovE[13])

    @pl.when(wid == 0)
    def _():
        sync_slab(wid + 61 * NW, ovE[13], ovE[14])

    @pl.when(wid == 1)
    def _():
        for half in (0, 1):
            pltpu.sync_copy(ins[half].at[:, pl.ds(1953 * SW, 64)],
                            slabe_v.at[half])
        apply_updates(slabe_v, ovE[13], ovE[14])
        for half in (0, 1):
            pltpu.sync_copy(slabe_v.at[half],
                            outs[half].at[:, pl.ds(1953 * SW, 64)])


_apply = pl.kernel(
    _apply_body,
    out_type=(jax.ShapeDtypeStruct((DIM, N_ROWS), jnp.float32),
              jax.ShapeDtypeStruct((DIM, N_ROWS), jnp.float32)),
    mesh=_mesh,
    compiler_params=pltpu.CompilerParams(needs_layout_passes=False),
    scratch_types=[
        pltpu.VMEM((1024,), jnp.int32),
        pltpu.VMEM((CAP,), jnp.int32),
        pltpu.VMEM((CAP,), jnp.int32),
        pltpu.VMEM((256,), jnp.int32),
        pltpu.VMEM((CAP,), jnp.int32),
        pltpu.VMEM((CAP, MW), jnp.float32),
        pltpu.VMEM((3, 2, DIM, SW), jnp.float32),
        pltpu.VMEM((2, DIM, 64), jnp.float32),
        pltpu.VMEM((DIM,), jnp.float32),
    ] + [pltpu.SemaphoreType.DMA] * 9,
)


def kernel(mL, mT, idx, zL, zT, alpha):
    a_vec = jnp.full((DIM,), alpha, jnp.float32)
    msg = _prep(zL.T, zT.T, a_vec)
    outLt, outTt = _apply(idx, mL.T, mT.T, msg, a_vec)
    return outLt.T, outTt.T
